# Initial kernel scaffold; baseline (speedup 1.0000x reference)
import jax
import jax.numpy as jnp
from jax.experimental import pallas as pl


def kernel(z, W):
    raise NotImplementedError("write your pallas kernel here")


# speedup vs baseline: 13.6612x; 13.6612x over previous
import jax
import jax.numpy as jnp
from jax.experimental import pallas as pl

_NUM_CODES = 512
_LATENT = 64


def _body(x_ref, w_ref, soft_ref, hard_ref, idx_ref):
    x = x_ref[0]
    w = w_ref[0]
    d = jnp.abs(x - w)
    mind = jnp.min(d, axis=0, keepdims=True)
    kio = jax.lax.broadcasted_iota(jnp.int32, d.shape, 0)
    idx = jnp.min(jnp.where(d == mind, kio, _NUM_CODES), axis=0, keepdims=True)
    e = jnp.exp(mind - d)
    se = jnp.sum(e, axis=0, keepdims=True)
    sw = jnp.sum(e * w, axis=0, keepdims=True)
    soft_ref[0] = sw / se
    hard_ref[0] = jnp.sum(jnp.where(kio == idx, w, 0.0), axis=0, keepdims=True)
    idx_ref[0] = idx


def kernel(z, W):
    B, C, H, Wd = z.shape
    E = B * H * Wd
    K = W.shape[1]
    X = jnp.transpose(z, (1, 0, 2, 3)).reshape(C, 1, E)
    Wc = W.reshape(C, K, 1)

    grid = (C,)
    soft, hard, idx = pl.pallas_call(
        _body,
        grid=grid,
        in_specs=[
            pl.BlockSpec((1, 1, E), lambda c: (c, 0, 0)),
            pl.BlockSpec((1, K, 1), lambda c: (c, 0, 0)),
        ],
        out_specs=[
            pl.BlockSpec((1, 1, E), lambda c: (c, 0, 0)),
            pl.BlockSpec((1, 1, E), lambda c: (c, 0, 0)),
            pl.BlockSpec((1, 1, E), lambda c: (c, 0, 0)),
        ],
        out_shape=[
            jax.ShapeDtypeStruct((C, 1, E), jnp.float32),
            jax.ShapeDtypeStruct((C, 1, E), jnp.float32),
            jax.ShapeDtypeStruct((C, 1, E), jnp.int32),
        ],
    )(X, Wc)

    def back(a):
        return jnp.transpose(a.reshape(C, B, H, Wd), (1, 2, 3, 0))

    return (back(soft), back(hard), back(idx))


# speedup vs baseline: 29.6393x; 2.1696x over previous
import functools

import jax
import jax.numpy as jnp
from jax import lax
from jax.experimental import pallas as pl
from jax.experimental.pallas import tpu as pltpu
from jax.experimental.pallas import tpu_sc as plsc

_NUM_CODES = 512
_LATENT = 64
_E = 2304
_LANES = 16
_GROUPS = _E // _LANES
_TROW = 528


def _build_tables(W):
    C, K = W.shape
    order = jnp.argsort(W, axis=1, stable=True).astype(jnp.int32)
    ws = jnp.take_along_axis(W, order, axis=1)
    iota = jnp.broadcast_to(jnp.arange(K, dtype=jnp.int32), (C, K))
    is_new = jnp.concatenate(
        [jnp.ones((C, 1), bool), ws[:, 1:] != ws[:, :-1]], axis=1)
    run_start = lax.cummax(jnp.where(is_new, iota, 0), axis=1)
    fidx = jnp.take_along_axis(order, run_start, axis=1)
    expw, expnw = jnp.exp(ws), jnp.exp(-ws)
    z1 = jnp.zeros((C, 1), jnp.float32)
    inf = jnp.full((C, 1), jnp.inf, jnp.float32)
    zi = jnp.zeros((C, 1), jnp.int32)
    rows = [
        jnp.concatenate([-inf, ws], axis=1),
        jnp.concatenate([ws, inf], axis=1),
        jnp.concatenate([z1, jnp.cumsum(expw, axis=1)], axis=1),
        jnp.concatenate([z1, jnp.cumsum(expw * ws, axis=1)], axis=1),
        jnp.concatenate(
            [jnp.flip(jnp.cumsum(jnp.flip(expnw, 1), 1), 1), z1], axis=1),
        jnp.concatenate(
            [jnp.flip(jnp.cumsum(jnp.flip(expnw * ws, 1), 1), 1), z1], axis=1),
        jax.lax.bitcast_convert_type(
            jnp.concatenate([zi, fidx], axis=1), jnp.float32),
        jax.lax.bitcast_convert_type(
            jnp.concatenate([fidx, zi], axis=1), jnp.float32),
    ]
    pads = [0.0, jnp.inf, 0.0, 0.0, 0.0, 0.0, 0.0, 0.0]
    padded = [
        jnp.concatenate(
            [r, jnp.full((C, _TROW - (K + 1)), p, jnp.float32)], axis=1)
        for r, p in zip(rows, pads)
    ]
    return jnp.concatenate(padded, axis=1)


def _make_sc_call(C):
    mesh = plsc.VectorSubcoreMesh(core_axis_name="c", subcore_axis_name="s")
    n_workers = 32
    chans_per_worker = C // n_workers

    @functools.partial(
        pl.kernel,
        mesh=mesh,
        out_type=[
            jax.ShapeDtypeStruct((C, _E), jnp.float32),
            jax.ShapeDtypeStruct((C, _E), jnp.float32),
            jax.ShapeDtypeStruct((C, _E), jnp.int32),
        ],
        scratch_types=[
            pltpu.VMEM((_E,), jnp.float32),
            pltpu.VMEM((8 * _TROW,), jnp.float32),
            pltpu.VMEM((_E,), jnp.float32),
            pltpu.VMEM((_E,), jnp.float32),
            pltpu.VMEM((_E,), jnp.int32),
        ],
        compiler_params=pltpu.CompilerParams(needs_layout_passes=False),
    )
    def sc_quantize(x_hbm, t_hbm, soft_hbm, hard_hbm, idx_hbm,
                    x_v, t_v, soft_v, hard_v, idx_v):
        wid = lax.axis_index("s") * 2 + lax.axis_index("c")

        for t in range(chans_per_worker):
            c = wid * chans_per_worker + t
            pltpu.sync_copy(x_hbm.at[c], x_v)
            pltpu.sync_copy(t_hbm.at[c], t_v)

            def group(i, carry):
                base = i * _LANES
                vv = x_v[pl.ds(base, _LANES)]
                j = jnp.zeros((_LANES,), jnp.int32)
                for step in (256, 128, 64, 32, 16, 8, 4, 2, 1):
                    probe = plsc.load_gather(t_v, [j + (_TROW + step - 1)])
                    j = jnp.where(probe < vv, j + step, j)
                probe = plsc.load_gather(t_v, [j + _TROW])
                j = jnp.where(probe < vv, j + 1, j)

                wl = plsc.load_gather(t_v, [j])
                wr = plsc.load_gather(t_v, [j + _TROW])
                a = plsc.load_gather(t_v, [j + 2 * _TROW])
                aw = plsc.load_gather(t_v, [j + 3 * _TROW])
                b = plsc.load_gather(t_v, [j + 4 * _TROW])
                bw = plsc.load_gather(t_v, [j + 5 * _TROW])
                fl = plsc.bitcast(
                    plsc.load_gather(t_v, [j + 6 * _TROW]), jnp.int32)
                fr = plsc.bitcast(
                    plsc.load_gather(t_v, [j + 7 * _TROW]), jnp.int32)

                ev = jnp.exp(vv)
                env = jnp.exp(-vv)
                soft = (env * aw + ev * bw) / (env * a + ev * b)
                dl = vv - wl
                dr = wr - vv
                pick_l = (dl < dr) | ((dl == dr) & (fl < fr))
                soft_v[pl.ds(base, _LANES)] = soft
                hard_v[pl.ds(base, _LANES)] = jnp.where(pick_l, wl, wr)
                idx_v[pl.ds(base, _LANES)] = jnp.where(pick_l, fl, fr)
                return carry

            lax.fori_loop(0, _GROUPS, group, 0)
            pltpu.sync_copy(soft_v, soft_hbm.at[c])
            pltpu.sync_copy(hard_v, hard_hbm.at[c])
            pltpu.sync_copy(idx_v, idx_hbm.at[c])

    return sc_quantize


def kernel(z, W):
    B, C, H, Wd = z.shape
    X = jnp.transpose(z, (1, 0, 2, 3)).reshape(C, _E)
    T = _build_tables(W)
    soft, hard, idx = _make_sc_call(C)(X, T)

    def back(a):
        return jnp.transpose(a.reshape(C, B, H, Wd), (1, 2, 3, 0))

    return (back(soft), back(hard), back(idx))


# speedup vs baseline: 30.3332x; 1.0234x over previous
import functools

import jax
import jax.numpy as jnp
from jax import lax
from jax.experimental import pallas as pl
from jax.experimental.pallas import tpu as pltpu
from jax.experimental.pallas import tpu_sc as plsc

_NUM_CODES = 512
_LATENT = 64
_E = 2304
_LANES = 16
_GROUPS = _E // _LANES
_TROW = 528


def _build_tables(W):
    C, K = W.shape
    iota = jnp.broadcast_to(jnp.arange(K, dtype=jnp.int32), (C, K))
    ws, order = lax.sort((W, iota), dimension=1, num_keys=1, is_stable=True)
    is_new = jnp.concatenate(
        [jnp.ones((C, 1), bool), ws[:, 1:] != ws[:, :-1]], axis=1)
    run_start = lax.cummax(jnp.where(is_new, iota, 0), axis=1)
    onehot = (run_start[:, :, None] == iota[:, None, :]).astype(jnp.float32)
    fidx = jnp.einsum(
        "cpq,cq->cp", onehot, order.astype(jnp.float32),
        preferred_element_type=jnp.float32).astype(jnp.int32)
    expw, expnw = jnp.exp(ws), jnp.exp(-ws)
    z1 = jnp.zeros((C, 1), jnp.float32)
    inf = jnp.full((C, 1), jnp.inf, jnp.float32)
    zi = jnp.zeros((C, 1), jnp.int32)
    rows = [
        jnp.concatenate([-inf, ws], axis=1),
        jnp.concatenate([ws, inf], axis=1),
        jnp.concatenate([z1, jnp.cumsum(expw, axis=1)], axis=1),
        jnp.concatenate([z1, jnp.cumsum(expw * ws, axis=1)], axis=1),
        jnp.concatenate(
            [jnp.flip(jnp.cumsum(jnp.flip(expnw, 1), 1), 1), z1], axis=1),
        jnp.concatenate(
            [jnp.flip(jnp.cumsum(jnp.flip(expnw * ws, 1), 1), 1), z1], axis=1),
        jax.lax.bitcast_convert_type(
            jnp.concatenate([zi, fidx], axis=1), jnp.float32),
        jax.lax.bitcast_convert_type(
            jnp.concatenate([fidx, zi], axis=1), jnp.float32),
    ]
    pads = [0.0, jnp.inf, 0.0, 0.0, 0.0, 0.0, 0.0, 0.0]
    padded = [
        jnp.concatenate(
            [r, jnp.full((C, _TROW - (K + 1)), p, jnp.float32)], axis=1)
        for r, p in zip(rows, pads)
    ]
    return jnp.concatenate(padded, axis=1)


def _make_sc_call(C):
    mesh = plsc.VectorSubcoreMesh(core_axis_name="c", subcore_axis_name="s")
    n_workers = 32
    chans_per_worker = C // n_workers

    @functools.partial(
        pl.kernel,
        mesh=mesh,
        out_type=[
            jax.ShapeDtypeStruct((C, _E), jnp.float32),
            jax.ShapeDtypeStruct((C, _E), jnp.float32),
            jax.ShapeDtypeStruct((C, _E), jnp.int32),
        ],
        scratch_types=[
            pltpu.VMEM((_E,), jnp.float32),
            pltpu.VMEM((8 * _TROW,), jnp.float32),
            pltpu.VMEM((_E,), jnp.float32),
            pltpu.VMEM((_E,), jnp.float32),
            pltpu.VMEM((_E,), jnp.int32),
        ],
        compiler_params=pltpu.CompilerParams(needs_layout_passes=False),
    )
    def sc_quantize(x_hbm, t_hbm, soft_hbm, hard_hbm, idx_hbm,
                    x_v, t_v, soft_v, hard_v, idx_v):
        wid = lax.axis_index("s") * 2 + lax.axis_index("c")

        for t in range(chans_per_worker):
            c = wid * chans_per_worker + t
            pltpu.sync_copy(x_hbm.at[c], x_v)
            pltpu.sync_copy(t_hbm.at[c], t_v)

            def group(i, carry):
                base = i * _LANES
                vv = x_v[pl.ds(base, _LANES)]
                j = jnp.zeros((_LANES,), jnp.int32)
                for step in (256, 128, 64, 32, 16, 8, 4, 2, 1):
                    probe = plsc.load_gather(t_v, [j + (_TROW + step - 1)])
                    j = jnp.where(probe < vv, j + step, j)
                probe = plsc.load_gather(t_v, [j + _TROW])
                j = jnp.where(probe < vv, j + 1, j)

                wl = plsc.load_gather(t_v, [j])
                wr = plsc.load_gather(t_v, [j + _TROW])
                a = plsc.load_gather(t_v, [j + 2 * _TROW])
                aw = plsc.load_gather(t_v, [j + 3 * _TROW])
                b = plsc.load_gather(t_v, [j + 4 * _TROW])
                bw = plsc.load_gather(t_v, [j + 5 * _TROW])
                fl = plsc.bitcast(
                    plsc.load_gather(t_v, [j + 6 * _TROW]), jnp.int32)
                fr = plsc.bitcast(
                    plsc.load_gather(t_v, [j + 7 * _TROW]), jnp.int32)

                ev = jnp.exp(vv)
                env = jnp.exp(-vv)
                soft = (env * aw + ev * bw) / (env * a + ev * b)
                dl = vv - wl
                dr = wr - vv
                pick_l = (dl < dr) | ((dl == dr) & (fl < fr))
                soft_v[pl.ds(base, _LANES)] = soft
                hard_v[pl.ds(base, _LANES)] = jnp.where(pick_l, wl, wr)
                idx_v[pl.ds(base, _LANES)] = jnp.where(pick_l, fl, fr)
                return carry

            lax.fori_loop(0, _GROUPS, group, 0)
            pltpu.sync_copy(soft_v, soft_hbm.at[c])
            pltpu.sync_copy(hard_v, hard_hbm.at[c])
            pltpu.sync_copy(idx_v, idx_hbm.at[c])

    return sc_quantize


def kernel(z, W):
    B, C, H, Wd = z.shape
    X = jnp.transpose(z, (1, 0, 2, 3)).reshape(C, _E)
    T = _build_tables(W)
    soft, hard, idx = _make_sc_call(C)(X, T)

    def back(a):
        return jnp.transpose(a.reshape(C, B, H, Wd), (1, 2, 3, 0))

    return (back(soft), back(hard), back(idx))


# speedup vs baseline: 34.9350x; 1.1517x over previous
import functools

import jax
import jax.numpy as jnp
from jax import lax
from jax.experimental import pallas as pl
from jax.experimental.pallas import tpu as pltpu
from jax.experimental.pallas import tpu_sc as plsc

_NUM_CODES = 512
_LATENT = 64
_E = 2304
_LANES = 16
_GROUPS = _E // _LANES
_TROW = 528


def _build_tables(W):
    C, K = W.shape
    iota = jnp.broadcast_to(jnp.arange(K, dtype=jnp.int32), (C, K))
    ws, order = lax.sort((W, iota), dimension=1, num_keys=1, is_stable=True)
    is_new = jnp.concatenate(
        [jnp.ones((C, 1), bool), ws[:, 1:] != ws[:, :-1]], axis=1)
    tagged = jnp.where(is_new, (iota << 9) | order, 0)
    fidx = lax.cummax(tagged, axis=1) & (2 ** 9 - 1)
    expw, expnw = jnp.exp(ws), jnp.exp(-ws)
    z1 = jnp.zeros((C, 1), jnp.float32)
    inf = jnp.full((C, 1), jnp.inf, jnp.float32)
    zi = jnp.zeros((C, 1), jnp.int32)
    rows = [
        jnp.concatenate([-inf, ws], axis=1),
        jnp.concatenate([ws, inf], axis=1),
        jnp.concatenate([z1, jnp.cumsum(expw, axis=1)], axis=1),
        jnp.concatenate([z1, jnp.cumsum(expw * ws, axis=1)], axis=1),
        jnp.concatenate(
            [jnp.flip(jnp.cumsum(jnp.flip(expnw, 1), 1), 1), z1], axis=1),
        jnp.concatenate(
            [jnp.flip(jnp.cumsum(jnp.flip(expnw * ws, 1), 1), 1), z1], axis=1),
        jax.lax.bitcast_convert_type(
            jnp.concatenate([zi, fidx], axis=1), jnp.float32),
        jax.lax.bitcast_convert_type(
            jnp.concatenate([fidx, zi], axis=1), jnp.float32),
    ]
    pads = [0.0, jnp.inf, 0.0, 0.0, 0.0, 0.0, 0.0, 0.0]
    padded = [
        jnp.concatenate(
            [r, jnp.full((C, _TROW - (K + 1)), p, jnp.float32)], axis=1)
        for r, p in zip(rows, pads)
    ]
    return jnp.concatenate(padded, axis=1)


def _make_sc_call(C):
    mesh = plsc.VectorSubcoreMesh(core_axis_name="c", subcore_axis_name="s")
    n_workers = 32
    chans_per_worker = C // n_workers

    @functools.partial(
        pl.kernel,
        mesh=mesh,
        out_type=[
            jax.ShapeDtypeStruct((C, _E), jnp.float32),
            jax.ShapeDtypeStruct((C, _E), jnp.float32),
            jax.ShapeDtypeStruct((C, _E), jnp.int32),
        ],
        scratch_types=[
            pltpu.VMEM((_E,), jnp.float32),
            pltpu.VMEM((8 * _TROW,), jnp.float32),
            pltpu.VMEM((_E,), jnp.float32),
            pltpu.VMEM((_E,), jnp.float32),
            pltpu.VMEM((_E,), jnp.int32),
        ],
        compiler_params=pltpu.CompilerParams(needs_layout_passes=False),
    )
    def sc_quantize(x_hbm, t_hbm, soft_hbm, hard_hbm, idx_hbm,
                    x_v, t_v, soft_v, hard_v, idx_v):
        wid = lax.axis_index("s") * 2 + lax.axis_index("c")

        for t in range(chans_per_worker):
            c = wid * chans_per_worker + t
            pltpu.sync_copy(x_hbm.at[c], x_v)
            pltpu.sync_copy(t_hbm.at[c], t_v)

            def group(i, carry):
                base = i * _LANES
                vv = x_v[pl.ds(base, _LANES)]
                j = jnp.zeros((_LANES,), jnp.int32)
                for step in (256, 128, 64, 32, 16, 8, 4, 2, 1):
                    probe = plsc.load_gather(t_v, [j + (_TROW + step - 1)])
                    j = jnp.where(probe < vv, j + step, j)
                probe = plsc.load_gather(t_v, [j + _TROW])
                j = jnp.where(probe < vv, j + 1, j)

                wl = plsc.load_gather(t_v, [j])
                wr = plsc.load_gather(t_v, [j + _TROW])
                a = plsc.load_gather(t_v, [j + 2 * _TROW])
                aw = plsc.load_gather(t_v, [j + 3 * _TROW])
                b = plsc.load_gather(t_v, [j + 4 * _TROW])
                bw = plsc.load_gather(t_v, [j + 5 * _TROW])
                fl = plsc.bitcast(
                    plsc.load_gather(t_v, [j + 6 * _TROW]), jnp.int32)
                fr = plsc.bitcast(
                    plsc.load_gather(t_v, [j + 7 * _TROW]), jnp.int32)

                ev = jnp.exp(vv)
                env = jnp.exp(-vv)
                soft = (env * aw + ev * bw) / (env * a + ev * b)
                dl = vv - wl
                dr = wr - vv
                pick_l = (dl < dr) | ((dl == dr) & (fl < fr))
                soft_v[pl.ds(base, _LANES)] = soft
                hard_v[pl.ds(base, _LANES)] = jnp.where(pick_l, wl, wr)
                idx_v[pl.ds(base, _LANES)] = jnp.where(pick_l, fl, fr)
                return carry

            lax.fori_loop(0, _GROUPS, group, 0)
            pltpu.sync_copy(soft_v, soft_hbm.at[c])
            pltpu.sync_copy(hard_v, hard_hbm.at[c])
            pltpu.sync_copy(idx_v, idx_hbm.at[c])

    return sc_quantize


def kernel(z, W):
    B, C, H, Wd = z.shape
    X = jnp.transpose(z, (1, 0, 2, 3)).reshape(C, _E)
    T = _build_tables(W)
    soft, hard, idx = _make_sc_call(C)(X, T)

    def back(a):
        return jnp.transpose(a.reshape(C, B, H, Wd), (1, 2, 3, 0))

    return (back(soft), back(hard), back(idx))


# speedup vs baseline: 46.9211x; 1.3431x over previous
import functools

import jax
import jax.numpy as jnp
from jax import lax
from jax.experimental import pallas as pl
from jax.experimental.pallas import tpu as pltpu
from jax.experimental.pallas import tpu_sc as plsc

_NUM_CODES = 512
_LATENT = 64
_E = 2304
_LANES = 16
_GROUPS = _E // _LANES
_TROW = 528


def _build_tables(W):
    C, K = W.shape
    iota = jnp.broadcast_to(jnp.arange(K, dtype=jnp.int32), (C, K))
    ws, order = lax.sort((W, iota), dimension=1, num_keys=1, is_stable=True)
    is_new = jnp.concatenate(
        [jnp.ones((C, 1), bool), ws[:, 1:] != ws[:, :-1]], axis=1)
    tagged = jnp.where(is_new, (iota << 9) | order, 0)
    fidx = lax.cummax(tagged, axis=1) & (2 ** 9 - 1)
    expw, expnw = jnp.exp(ws), jnp.exp(-ws)
    z1 = jnp.zeros((C, 1), jnp.float32)
    inf = jnp.full((C, 1), jnp.inf, jnp.float32)
    zi = jnp.zeros((C, 1), jnp.int32)
    rows = [
        jnp.concatenate([-inf, ws], axis=1),
        jnp.concatenate([ws, inf], axis=1),
        jnp.concatenate([z1, jnp.cumsum(expw, axis=1)], axis=1),
        jnp.concatenate([z1, jnp.cumsum(expw * ws, axis=1)], axis=1),
        jnp.concatenate(
            [jnp.flip(jnp.cumsum(jnp.flip(expnw, 1), 1), 1), z1], axis=1),
        jnp.concatenate(
            [jnp.flip(jnp.cumsum(jnp.flip(expnw * ws, 1), 1), 1), z1], axis=1),
        jax.lax.bitcast_convert_type(
            jnp.concatenate([zi, fidx], axis=1), jnp.float32),
        jax.lax.bitcast_convert_type(
            jnp.concatenate([fidx, zi], axis=1), jnp.float32),
    ]
    pads = [0.0, jnp.inf, 0.0, 0.0, 0.0, 0.0, 0.0, 0.0]
    padded = [
        jnp.concatenate(
            [r, jnp.full((C, _TROW - (K + 1)), p, jnp.float32)], axis=1)
        for r, p in zip(rows, pads)
    ]
    return jnp.concatenate(padded, axis=1)


def _make_sc_call(C):
    mesh = plsc.VectorSubcoreMesh(core_axis_name="c", subcore_axis_name="s")
    n_workers = 32
    chans_per_worker = C // n_workers

    @functools.partial(
        pl.kernel,
        mesh=mesh,
        out_type=[
            jax.ShapeDtypeStruct((C, _E), jnp.float32),
            jax.ShapeDtypeStruct((C, _E), jnp.float32),
            jax.ShapeDtypeStruct((C, _E), jnp.int32),
        ],
        scratch_types=(
            [pltpu.VMEM((_E,), jnp.float32)] * 2
            + [pltpu.VMEM((8 * _TROW,), jnp.float32)] * 2
            + [pltpu.VMEM((_E,), jnp.float32)] * 4
            + [pltpu.VMEM((_E,), jnp.int32)] * 2
            + [pltpu.SemaphoreType.DMA] * 3
        ),
        compiler_params=pltpu.CompilerParams(needs_layout_passes=False),
    )
    def sc_quantize(x_hbm, t_hbm, soft_hbm, hard_hbm, idx_hbm,
                    x0, x1, t0, t1, s0, s1, h0, h1, i0, i1,
                    sin0, sin1, sout):
        wid = lax.axis_index("s") * 2 + lax.axis_index("c")
        c0 = wid * chans_per_worker

        bufs = [(x0, t0, s0, h0, i0), (x1, t1, s1, h1, i1)]
        sins = [sin0, sin1]
        dins = []
        for t in range(chans_per_worker):
            dins.append((
                pltpu.async_copy(x_hbm.at[c0 + t], bufs[t][0], sins[t]),
                pltpu.async_copy(t_hbm.at[c0 + t], bufs[t][1], sins[t]),
            ))

        douts = []
        for t in range(chans_per_worker):
            dins[t][0].wait()
            dins[t][1].wait()
            xv, tv, sv, hv, iv = bufs[t]

            @plsc.parallel_loop(0, _GROUPS, unroll=8)
            def group(i):
                base = i * _LANES
                vv = xv[pl.ds(base, _LANES)]
                j = jnp.zeros((_LANES,), jnp.int32)
                for step in (256, 128, 64, 32, 16, 8, 4, 2, 1):
                    probe = plsc.load_gather(tv, [j + (_TROW + step - 1)])
                    j = jnp.where(probe < vv, j + step, j)
                probe = plsc.load_gather(tv, [j + _TROW])
                j = jnp.where(probe < vv, j + 1, j)

                wl = plsc.load_gather(tv, [j])
                wr = plsc.load_gather(tv, [j + _TROW])
                a = plsc.load_gather(tv, [j + 2 * _TROW])
                aw = plsc.load_gather(tv, [j + 3 * _TROW])
                b = plsc.load_gather(tv, [j + 4 * _TROW])
                bw = plsc.load_gather(tv, [j + 5 * _TROW])
                fl = plsc.bitcast(
                    plsc.load_gather(tv, [j + 6 * _TROW]), jnp.int32)
                fr = plsc.bitcast(
                    plsc.load_gather(tv, [j + 7 * _TROW]), jnp.int32)

                ev = jnp.exp(vv)
                env = jnp.exp(-vv)
                soft = (env * aw + ev * bw) / (env * a + ev * b)
                dl = vv - wl
                dr = wr - vv
                pick_l = (dl < dr) | ((dl == dr) & (fl < fr))
                sv[pl.ds(base, _LANES)] = soft
                hv[pl.ds(base, _LANES)] = jnp.where(pick_l, wl, wr)
                iv[pl.ds(base, _LANES)] = jnp.where(pick_l, fl, fr)

            c = c0 + t
            douts.append(pltpu.async_copy(sv, soft_hbm.at[c], sout))
            douts.append(pltpu.async_copy(hv, hard_hbm.at[c], sout))
            douts.append(pltpu.async_copy(iv, idx_hbm.at[c], sout))

        for d in douts:
            d.wait()

    return sc_quantize


def kernel(z, W):
    B, C, H, Wd = z.shape
    X = jnp.transpose(z, (1, 0, 2, 3)).reshape(C, _E)
    T = _build_tables(W)
    soft, hard, idx = _make_sc_call(C)(X, T)

    def back(a):
        return jnp.transpose(a.reshape(C, B, H, Wd), (1, 2, 3, 0))

    return (back(soft), back(hard), back(idx))


# speedup vs baseline: 49.9278x; 1.0641x over previous
import functools

import jax
import jax.numpy as jnp
from jax import lax
from jax.experimental import pallas as pl
from jax.experimental.pallas import tpu as pltpu
from jax.experimental.pallas import tpu_sc as plsc

_NUM_CODES = 512
_LATENT = 64
_E = 2304
_LANES = 16
_GROUPS = _E // _LANES
_TROW = 528


def _build_tables(W):
    C, K = W.shape
    iota = jnp.broadcast_to(jnp.arange(K, dtype=jnp.int32), (C, K))
    ws, order = lax.sort((W, iota), dimension=1, num_keys=1, is_stable=True)
    is_new = jnp.concatenate(
        [jnp.ones((C, 1), bool), ws[:, 1:] != ws[:, :-1]], axis=1)
    tagged = jnp.where(is_new, (iota << 9) | order, 0)
    fidx = lax.cummax(tagged, axis=1) & (2 ** 9 - 1)
    expw, expnw = jnp.exp(ws), jnp.exp(-ws)
    z1 = jnp.zeros((C, 1), jnp.float32)
    inf = jnp.full((C, 1), jnp.inf, jnp.float32)
    zi = jnp.zeros((C, 1), jnp.int32)
    rows = [
        jnp.concatenate([-inf, ws], axis=1),
        jnp.concatenate([ws, inf], axis=1),
        jnp.concatenate([z1, jnp.cumsum(expw, axis=1)], axis=1),
        jnp.concatenate([z1, jnp.cumsum(expw * ws, axis=1)], axis=1),
        jnp.concatenate(
            [jnp.flip(jnp.cumsum(jnp.flip(expnw, 1), 1), 1), z1], axis=1),
        jnp.concatenate(
            [jnp.flip(jnp.cumsum(jnp.flip(expnw * ws, 1), 1), 1), z1], axis=1),
        jax.lax.bitcast_convert_type(
            jnp.concatenate([zi, fidx], axis=1), jnp.float32),
        jax.lax.bitcast_convert_type(
            jnp.concatenate([fidx, zi], axis=1), jnp.float32),
    ]
    pads = [0.0, jnp.inf, 0.0, 0.0, 0.0, 0.0, 0.0, 0.0]
    padded = [
        jnp.concatenate(
            [r, jnp.full((C, _TROW - (K + 1)), p, jnp.float32)], axis=1)
        for r, p in zip(rows, pads)
    ]
    return jnp.concatenate(padded, axis=1)


def _make_sc_call(C):
    mesh = plsc.VectorSubcoreMesh(core_axis_name="c", subcore_axis_name="s")
    n_workers = 32
    chans_per_worker = C // n_workers

    @functools.partial(
        pl.kernel,
        mesh=mesh,
        out_type=[
            jax.ShapeDtypeStruct((C, _E), jnp.float32),
            jax.ShapeDtypeStruct((C, _E), jnp.float32),
            jax.ShapeDtypeStruct((C, _E), jnp.int32),
        ],
        scratch_types=(
            [pltpu.VMEM((_E,), jnp.float32)] * 2
            + [pltpu.VMEM((8 * _TROW,), jnp.float32)] * 2
            + [pltpu.VMEM((_E,), jnp.float32)] * 4
            + [pltpu.VMEM((_E,), jnp.int32)] * 2
            + [pltpu.SemaphoreType.DMA] * 3
        ),
        compiler_params=pltpu.CompilerParams(needs_layout_passes=False),
    )
    def sc_quantize(x_hbm, t_hbm, soft_hbm, hard_hbm, idx_hbm,
                    x0, x1, t0, t1, s0, s1, h0, h1, i0, i1,
                    sin0, sin1, sout):
        wid = lax.axis_index("s") * 2 + lax.axis_index("c")
        c0 = wid * chans_per_worker

        bufs = [(x0, t0, s0, h0, i0), (x1, t1, s1, h1, i1)]
        sins = [sin0, sin1]
        dins = []
        for t in range(chans_per_worker):
            dins.append((
                pltpu.async_copy(x_hbm.at[c0 + t], bufs[t][0], sins[t]),
                pltpu.async_copy(t_hbm.at[c0 + t], bufs[t][1], sins[t]),
            ))

        douts = []
        for t in range(chans_per_worker):
            dins[t][0].wait()
            dins[t][1].wait()
            xv, tv, sv, hv, iv = bufs[t]

            @plsc.parallel_loop(0, _GROUPS, unroll=16)
            def group(i):
                base = i * _LANES
                vv = xv[pl.ds(base, _LANES)]
                j = jnp.zeros((_LANES,), jnp.int32)
                for step in (256, 128, 64, 32, 16, 8, 4, 2, 1):
                    probe = plsc.load_gather(tv, [j + (_TROW + step - 1)])
                    j = jnp.where(probe < vv, j + step, j)
                probe = plsc.load_gather(tv, [j + _TROW])
                j = jnp.where(probe < vv, j + 1, j)

                wl = plsc.load_gather(tv, [j])
                wr = plsc.load_gather(tv, [j + _TROW])
                a = plsc.load_gather(tv, [j + 2 * _TROW])
                aw = plsc.load_gather(tv, [j + 3 * _TROW])
                b = plsc.load_gather(tv, [j + 4 * _TROW])
                bw = plsc.load_gather(tv, [j + 5 * _TROW])
                fl = plsc.bitcast(
                    plsc.load_gather(tv, [j + 6 * _TROW]), jnp.int32)
                fr = plsc.bitcast(
                    plsc.load_gather(tv, [j + 7 * _TROW]), jnp.int32)

                u = jnp.exp(2.0 * vv)
                soft = (aw + u * bw) / (a + u * b)
                dl = vv - wl
                dr = wr - vv
                pick_l = (dl < dr) | ((dl == dr) & (fl < fr))
                sv[pl.ds(base, _LANES)] = soft
                hv[pl.ds(base, _LANES)] = jnp.where(pick_l, wl, wr)
                iv[pl.ds(base, _LANES)] = jnp.where(pick_l, fl, fr)

            c = c0 + t
            douts.append(pltpu.async_copy(sv, soft_hbm.at[c], sout))
            douts.append(pltpu.async_copy(hv, hard_hbm.at[c], sout))
            douts.append(pltpu.async_copy(iv, idx_hbm.at[c], sout))

        for d in douts:
            d.wait()

    return sc_quantize


def kernel(z, W):
    B, C, H, Wd = z.shape
    X = jnp.transpose(z, (1, 0, 2, 3)).reshape(C, _E)
    T = _build_tables(W)
    soft, hard, idx = _make_sc_call(C)(X, T)

    def back(a):
        return jnp.transpose(a.reshape(C, B, H, Wd), (1, 2, 3, 0))

    return (back(soft), back(hard), back(idx))


# speedup vs baseline: 55.5683x; 1.1130x over previous
import functools

import jax
import jax.numpy as jnp
from jax import lax
from jax.experimental import pallas as pl
from jax.experimental.pallas import tpu as pltpu
from jax.experimental.pallas import tpu_sc as plsc

_NUM_CODES = 512
_LATENT = 64
_E = 2304
_LANES = 16
_GROUPS = _E // _LANES
_TROW = 528


def _build_tables(W):
    C, K = W.shape
    iota = jnp.broadcast_to(jnp.arange(K, dtype=jnp.int32), (C, K))
    ws, order = lax.sort((W, iota), dimension=1, num_keys=1, is_stable=True)
    is_new = jnp.concatenate(
        [jnp.ones((C, 1), bool), ws[:, 1:] != ws[:, :-1]], axis=1)
    tagged = jnp.where(is_new, (iota << 9) | order, 0)
    fidx = lax.cummax(tagged, axis=1) & (2 ** 9 - 1)
    expw, expnw = jnp.exp(ws), jnp.exp(-ws)
    kio = jnp.arange(K, dtype=jnp.int32)
    jio = jnp.arange(_TROW, dtype=jnp.int32)
    t_pre = (kio[:, None] < jio[None, :]).astype(jnp.float32)
    t_suf = (kio[:, None] >= jio[None, :]).astype(jnp.float32)
    e_pre = jnp.stack([expw, expw * ws], axis=1)
    e_suf = jnp.stack([expnw, expnw * ws], axis=1)
    hi = jax.lax.Precision.HIGHEST
    pre = jnp.einsum("cik,kj->cij", e_pre, t_pre, precision=hi)
    suf = jnp.einsum("cik,kj->cij", e_suf, t_suf, precision=hi)

    z1 = jnp.zeros((C, 1), jnp.float32)
    inf = jnp.full((C, 1), jnp.inf, jnp.float32)
    zi = jnp.zeros((C, 1), jnp.int32)
    pad = jnp.zeros((C, _TROW - (K + 1)), jnp.float32)
    rows = [
        jnp.concatenate([-inf, ws, pad], axis=1),
        jnp.concatenate([ws, inf, pad], axis=1),
        pre[:, 0],
        pre[:, 1],
        suf[:, 0],
        suf[:, 1],
        jax.lax.bitcast_convert_type(
            jnp.concatenate([zi, fidx, pad.astype(jnp.int32)], axis=1),
            jnp.float32),
        jax.lax.bitcast_convert_type(
            jnp.concatenate([fidx, zi, pad.astype(jnp.int32)], axis=1),
            jnp.float32),
    ]
    return jnp.concatenate(rows, axis=1)


def _make_sc_call(C):
    mesh = plsc.VectorSubcoreMesh(core_axis_name="c", subcore_axis_name="s")
    n_workers = 32
    chans_per_worker = C // n_workers

    @functools.partial(
        pl.kernel,
        mesh=mesh,
        out_type=[
            jax.ShapeDtypeStruct((C, _E), jnp.float32),
            jax.ShapeDtypeStruct((C, _E), jnp.float32),
            jax.ShapeDtypeStruct((C, _E), jnp.int32),
        ],
        scratch_types=(
            [pltpu.VMEM((_E,), jnp.float32)] * 2
            + [pltpu.VMEM((8 * _TROW,), jnp.float32)] * 2
            + [pltpu.VMEM((_E,), jnp.float32)] * 4
            + [pltpu.VMEM((_E,), jnp.int32)] * 2
            + [pltpu.SemaphoreType.DMA] * 3
        ),
        compiler_params=pltpu.CompilerParams(needs_layout_passes=False),
    )
    def sc_quantize(x_hbm, t_hbm, soft_hbm, hard_hbm, idx_hbm,
                    x0, x1, t0, t1, s0, s1, h0, h1, i0, i1,
                    sin0, sin1, sout):
        wid = lax.axis_index("s") * 2 + lax.axis_index("c")
        c0 = wid * chans_per_worker

        bufs = [(x0, t0, s0, h0, i0), (x1, t1, s1, h1, i1)]
        sins = [sin0, sin1]
        dins = []
        for t in range(chans_per_worker):
            dins.append((
                pltpu.async_copy(x_hbm.at[c0 + t], bufs[t][0], sins[t]),
                pltpu.async_copy(t_hbm.at[c0 + t], bufs[t][1], sins[t]),
            ))

        douts = []
        for t in range(chans_per_worker):
            dins[t][0].wait()
            dins[t][1].wait()
            xv, tv, sv, hv, iv = bufs[t]

            @plsc.parallel_loop(0, _GROUPS, unroll=16)
            def group(i):
                base = i * _LANES
                vv = xv[pl.ds(base, _LANES)]
                j = jnp.zeros((_LANES,), jnp.int32)
                for step in (256, 128, 64, 32, 16, 8, 4, 2, 1):
                    probe = plsc.load_gather(tv, [j + (_TROW + step - 1)])
                    j = jnp.where(probe < vv, j + step, j)
                probe = plsc.load_gather(tv, [j + _TROW])
                j = jnp.where(probe < vv, j + 1, j)

                wl = plsc.load_gather(tv, [j])
                wr = plsc.load_gather(tv, [j + _TROW])
                a = plsc.load_gather(tv, [j + 2 * _TROW])
                aw = plsc.load_gather(tv, [j + 3 * _TROW])
                b = plsc.load_gather(tv, [j + 4 * _TROW])
                bw = plsc.load_gather(tv, [j + 5 * _TROW])
                fl = plsc.bitcast(
                    plsc.load_gather(tv, [j + 6 * _TROW]), jnp.int32)
                fr = plsc.bitcast(
                    plsc.load_gather(tv, [j + 7 * _TROW]), jnp.int32)

                u = jnp.exp(2.0 * vv)
                soft = (aw + u * bw) / (a + u * b)
                dl = vv - wl
                dr = wr - vv
                pick_l = (dl < dr) | ((dl == dr) & (fl < fr))
                sv[pl.ds(base, _LANES)] = soft
                hv[pl.ds(base, _LANES)] = jnp.where(pick_l, wl, wr)
                iv[pl.ds(base, _LANES)] = jnp.where(pick_l, fl, fr)

            c = c0 + t
            douts.append(pltpu.async_copy(sv, soft_hbm.at[c], sout))
            douts.append(pltpu.async_copy(hv, hard_hbm.at[c], sout))
            douts.append(pltpu.async_copy(iv, idx_hbm.at[c], sout))

        for d in douts:
            d.wait()

    return sc_quantize


def kernel(z, W):
    B, C, H, Wd = z.shape
    X = jnp.transpose(z, (1, 0, 2, 3)).reshape(C, _E)
    T = _build_tables(W)
    soft, hard, idx = _make_sc_call(C)(X, T)

    def back(a):
        return jnp.transpose(a.reshape(C, B, H, Wd), (1, 2, 3, 0))

    return (back(soft), back(hard), back(idx))
